# baseline (speedup 1.0000x reference)
import functools
import jax
import jax.numpy as jnp
from jax import lax
from jax.experimental import pallas as pl
from jax.experimental.pallas import tpu as pltpu
from jax.experimental.pallas import tpu_sc as plsc

_N = 33554432
_NC = 2
_NS = 16
_NW = _NC * _NS
_PER = _N // _NW
_CH = 16384
_NCH = _PER // _CH
_D = 3
_NBUF = 6

@functools.lru_cache(maxsize=1)
def _get_sc_copy():
    mesh = plsc.VectorSubcoreMesh(core_axis_name="c", subcore_axis_name="s")

    @functools.partial(
        pl.kernel,
        mesh=mesh,
        out_type=jax.ShapeDtypeStruct((_N,), jnp.float32),
        scratch_types=[
            pltpu.VMEM((_NBUF, _CH), jnp.float32),
            pltpu.SemaphoreType.DMA((_NBUF,)),
            pltpu.SemaphoreType.DMA((_NBUF,)),
        ],
    )
    def _sc_copy(x_hbm, o_hbm, buf, gsem, ssem):
        wid = lax.axis_index("s") * _NC + lax.axis_index("c")
        base = wid * _PER

        def g_cp(i):
            return pltpu.make_async_copy(
                x_hbm.at[pl.ds(base + i * _CH, _CH)],
                buf.at[i % _NBUF],
                gsem.at[i % _NBUF],
            )

        def s_cp(i):
            return pltpu.make_async_copy(
                buf.at[i % _NBUF],
                o_hbm.at[pl.ds(base + i * _CH, _CH)],
                ssem.at[i % _NBUF],
            )

        for i in range(_D):
            g_cp(i).start()
        for i in range(_NCH):
            g_cp(i).wait()
            if i == 0:
                @pl.when(wid == 0)
                def _():
                    idx = lax.iota(jnp.int32, 16)
                    buf[0, 0:16] = jnp.where(idx == 0, 0.0, buf[0, 0:16])
            s_cp(i).start()
            j = i + _D
            if j < _NCH:
                if j >= _NBUF:
                    s_cp(j - _NBUF).wait()
                g_cp(j).start()
        for i in range(_NCH - _NBUF, _NCH):
            s_cp(i).wait()


    return _sc_copy

def kernel(x):
    return _get_sc_copy()(x)


# speedup vs baseline: 1.0028x; 1.0028x over previous
import functools
import jax
import jax.numpy as jnp
from jax import lax
from jax.experimental import pallas as pl
from jax.experimental.pallas import tpu as pltpu
from jax.experimental.pallas import tpu_sc as plsc

_N = 33554432
_NC = 2
_NS = 16
_NW = _NC * _NS
_PER = _N // _NW
_CH = 32768
_NCH = _PER // _CH
_D = 2
_NBUF = 3


@functools.lru_cache(maxsize=1)
def _get_sc_copy():
    mesh = plsc.VectorSubcoreMesh(core_axis_name="c", subcore_axis_name="s")

    @functools.partial(
        pl.kernel,
        mesh=mesh,
        out_type=jax.ShapeDtypeStruct((_N,), jnp.float32),
        scratch_types=(
            [pltpu.VMEM((_CH,), jnp.float32) for _ in range(_NBUF)]
            + [pltpu.SemaphoreType.DMA((_NBUF,)),
               pltpu.SemaphoreType.DMA((_NBUF,))]
        ),
    )
    def _sc_copy(x_hbm, o_hbm, *rest):
        bufs = rest[:_NBUF]
        gsem, ssem = rest[_NBUF], rest[_NBUF + 1]
        wid = lax.axis_index("s") * _NC + lax.axis_index("c")
        base = wid * _PER

        def g_cp(i):
            return pltpu.make_async_copy(
                x_hbm.at[pl.ds(base + i * _CH, _CH)],
                bufs[i % _NBUF],
                gsem.at[i % _NBUF],
            )

        def s_cp(i):
            return pltpu.make_async_copy(
                bufs[i % _NBUF],
                o_hbm.at[pl.ds(base + i * _CH, _CH)],
                ssem.at[i % _NBUF],
            )

        for i in range(_D):
            g_cp(i).start()
        for i in range(_NCH):
            g_cp(i).wait()
            if i == 0:
                @pl.when(wid == 0)
                def _():
                    idx = lax.iota(jnp.int32, 16)
                    bufs[0][0:16] = jnp.where(idx == 0, 0.0, bufs[0][0:16])
            s_cp(i).start()
            j = i + _D
            if j < _NCH:
                if j >= _NBUF:
                    s_cp(j - _NBUF).wait()
                g_cp(j).start()
        for i in range(_NCH - _NBUF, _NCH):
            s_cp(i).wait()

    return _sc_copy


def kernel(x):
    return _get_sc_copy()(x)
